# baseline (speedup 1.0000x reference)
import functools

import jax
import jax.numpy as jnp
from jax import lax
from jax.experimental import pallas as pl
from jax.experimental.pallas import tpu as pltpu
from jax.experimental.pallas import tpu_sc as plsc

_BATCH = 16384
_D = 32

_info = plsc.get_sparse_core_info()
_NC, _NS = _info.num_cores, _info.num_subcores
_NW = _NC * _NS
_BPW = _BATCH // _NW

_mesh = plsc.VectorSubcoreMesh(core_axis_name="c", subcore_axis_name="s")


@functools.partial(
    pl.kernel,
    mesh=_mesh,
    compiler_params=pltpu.CompilerParams(use_tc_tiling_on_sc=False,
                                         needs_layout_passes=False),
    out_type=[
        jax.ShapeDtypeStruct((_BATCH, _D), jnp.float32),
        jax.ShapeDtypeStruct((_BATCH, _D), jnp.float32),
        jax.ShapeDtypeStruct((_BATCH, _D), jnp.float32),
    ],
    scratch_types=[
        pltpu.VMEM((_BPW,), jnp.int32),
        pltpu.VMEM((_BPW,), jnp.int32),
        pltpu.VMEM((_BPW, 128), jnp.float32),
        pltpu.VMEM((_BPW, _D), jnp.float32),
        pltpu.SemaphoreType.DMA,
    ],
)
def _gather3(idx_u, idx_i, idx_t, t_u, t_i, t_t,
             out_u, out_i, out_t,
             iv, tv, fb, ob, sem):
    wid = lax.axis_index("s") * _NC + lax.axis_index("c")
    base = wid * _BPW
    lane = lax.iota(jnp.int32, 16)

    for idx, tbl, out in ((idx_u, t_u, out_u),
                          (idx_i, t_i, out_i),
                          (idx_t, t_t, out_t)):
        pltpu.sync_copy(idx.at[pl.ds(base, _BPW)], iv)

        def tv_body(g, _2):
            s = pl.ds(g * 16, 16)
            tv[s] = lax.shift_right_logical(iv[s], 2)
            return _2

        lax.fori_loop(0, _BPW // 16, tv_body, None)
        pltpu.async_copy(tbl.at[tv], fb, sem).wait()

        def extract_body(g, _2):
            rv = g * 16 + lane
            iv16 = iv[pl.ds(g * 16, 16)]
            cb = jnp.bitwise_and(iv16, 3) * _D
            for j in range(_D):
                jv = jnp.full((16,), j, jnp.int32)
                v = plsc.load_gather(fb, [rv, cb + j])
                plsc.store_scatter(ob, [rv, jv], v)
            return _2

        lax.fori_loop(0, _BPW // 16, extract_body, None)
        pltpu.sync_copy(ob, out.at[pl.ds(base, _BPW)])


def kernel(idx_user, idx_item, idx_tag, T_user, T_item, T_tag):
    out = _gather3(
        idx_user, idx_item, idx_tag,
        T_user.reshape(-1, 128),
        T_item.reshape(-1, 128),
        T_tag.reshape(-1, 128),
    )
    return (out[0], out[1], out[2])


# speedup vs baseline: 2.7146x; 2.7146x over previous
import functools

import jax
import jax.numpy as jnp
from jax import lax
from jax.experimental import pallas as pl
from jax.experimental.pallas import tpu as pltpu
from jax.experimental.pallas import tpu_sc as plsc

_BATCH = 16384
_D = 32

_info = plsc.get_sparse_core_info()
_NC, _NS = _info.num_cores, _info.num_subcores
_NW = _NC * _NS
_BPW = _BATCH // _NW

_mesh = plsc.VectorSubcoreMesh(core_axis_name="c", subcore_axis_name="s")

_SCRATCH = [
    pltpu.VMEM((_BPW,), jnp.int32),
    pltpu.VMEM((_BPW, _D), jnp.float32),
    pltpu.SemaphoreType.DMA,
    pltpu.SemaphoreType.DMA,
]
_OUT1 = jax.ShapeDtypeStruct((_BATCH, _D), jnp.float32)
_PARAMS = pltpu.CompilerParams(use_tc_tiling_on_sc=True,
                               needs_layout_passes=False)


def _gather_one(idx, tbl, out, iv, rb, sem, sem2):
    wid = lax.axis_index("s") * _NC + lax.axis_index("c")
    base = wid * _BPW
    pltpu.sync_copy(idx.at[pl.ds(base, _BPW)], iv)

    def issue_body(g, _2):
        v16 = iv[pl.ds(g * 16, 16)]
        for e in range(16):
            i = v16[e]
            t = lax.shift_right_logical(i, 3)
            s = jnp.bitwise_and(i, 7)
            pltpu.async_copy(tbl.at[t, s], rb.at[g * 16 + e], sem)
        return _2

    lax.fori_loop(0, _BPW // 16, issue_body, None)

    def drain_body(g, _2):
        for e in range(16):
            pltpu.make_async_copy(tbl.at[0, 0], rb.at[0], sem).wait()
        return _2

    lax.fori_loop(0, _BPW // 16, drain_body, None)
    pltpu.async_copy(rb, out.at[pl.ds(base, _BPW)], sem2).wait()


@functools.partial(
    pl.kernel, mesh=_mesh, compiler_params=_PARAMS,
    out_type=[_OUT1], scratch_types=_SCRATCH,
)
def _gather_user(idx_u, t_u, out_u, iv, rb, sem, sem2):
    _gather_one(idx_u, t_u, out_u, iv, rb, sem, sem2)


@functools.partial(
    pl.kernel, mesh=_mesh, compiler_params=_PARAMS,
    out_type=[_OUT1, _OUT1], scratch_types=_SCRATCH,
)
def _gather_small(idx_i, idx_t, t_i, t_t, out_i, out_t, iv, rb, sem, sem2):
    _gather_one(idx_i, t_i, out_i, iv, rb, sem, sem2)
    _gather_one(idx_t, t_t, out_t, iv, rb, sem, sem2)


def kernel(idx_user, idx_item, idx_tag, T_user, T_item, T_tag):
    out_u = _gather_user(idx_user, T_user.reshape(-1, 8, _D))
    out_i, out_t = _gather_small(idx_item, idx_tag,
                                 T_item.reshape(-1, 8, _D),
                                 T_tag.reshape(-1, 8, _D))
    return (out_u[0], out_i, out_t)
